# Initial kernel scaffold; baseline (speedup 1.0000x reference)
import jax
import jax.numpy as jnp
from jax.experimental import pallas as pl


def kernel(x, table):
    raise NotImplementedError("write your pallas kernel here")


# speedup vs baseline: 1.9622x; 1.9622x over previous
import functools

import jax
import jax.numpy as jnp
from jax import lax
from jax.experimental import pallas as pl
from jax.experimental.pallas import tpu as pltpu
from jax.experimental.pallas import tpu_sc as plsc

_B = 4 * 4096
_D = 1024
_NC = 2
_NS = 16
_NW = _NC * _NS
_B_PER_W = _B // _NW
_CHUNK = 32
_NCHUNKS = _B_PER_W // _CHUNK
_NBUF = 2


def _gather_body(table_hbm, idx_hbm, out_hbm, idx_v, buf0, buf1, sem0, sem1):
    wid = lax.axis_index("s") * _NC + lax.axis_index("c")
    base = wid * _B_PER_W
    pltpu.sync_copy(idx_hbm.at[pl.ds(base, _B_PER_W)], idx_v)

    bufs = (buf0, buf1)
    sems = (sem0, sem1)
    copies = [None] * _NCHUNKS
    copies[0] = pltpu.async_copy(
        table_hbm.at[idx_v.at[pl.ds(0, _CHUNK)]], bufs[0], sems[0])
    for g in range(_NCHUNKS):
        if g + 1 < _NCHUNKS:
            nb = (g + 1) % _NBUF
            copies[g + 1] = pltpu.async_copy(
                table_hbm.at[idx_v.at[pl.ds((g + 1) * _CHUNK, _CHUNK)]],
                bufs[nb], sems[nb])
        copies[g].wait()
        pltpu.sync_copy(bufs[g % _NBUF],
                        out_hbm.at[pl.ds(base + g * _CHUNK, _CHUNK)])


_sc_gather = functools.partial(
    pl.kernel,
    out_type=jax.ShapeDtypeStruct((_B, _D), jnp.float32),
    mesh=plsc.VectorSubcoreMesh(core_axis_name="c", subcore_axis_name="s"),
    scratch_types=[
        pltpu.VMEM((_B_PER_W,), jnp.int32),
        pltpu.VMEM((_CHUNK, _D), jnp.float32),
        pltpu.VMEM((_CHUNK, _D), jnp.float32),
        pltpu.SemaphoreType.DMA,
        pltpu.SemaphoreType.DMA,
    ],
)(_gather_body)


def kernel(x, table):
    idx = x.reshape(-1).astype(jnp.int32)
    out = _sc_gather(table, idx)
    return out.reshape(x.shape + (_D,))
